# Initial kernel scaffold; baseline (speedup 1.0000x reference)
import jax
import jax.numpy as jnp
from jax.experimental import pallas as pl


def kernel(xs, edge_index0, e_id0, edge_index1, e_id1, edge_weight, lipschitz, mu, std, W0n, W0r, W0l, b0, W1n, W1r, W1l, b1):
    raise NotImplementedError("write your pallas kernel here")


# speedup vs baseline: 2.5549x; 2.5549x over previous
import jax
import jax.numpy as jnp
from jax import lax
from jax.experimental import pallas as pl
from jax.experimental.pallas import tpu as pltpu
from jax.experimental.pallas import tpu_sc as plsc

N = 10000
E = 160000
NP = 10240
EP = 163840
GARBAGE_ROW = NP - 1
C = 128
BM = 256


def _tc1_body(mu_ref, std_ref, x_ref, lip_ref, w0n_ref, w0r_ref, w0l_ref,
              b0_ref, w1l_ref, y0_ref, r0_ref, l1_ref, xs3_ref):
    x = x_ref[...]
    lip = lip_ref[...]
    y0_ref[...] = jnp.dot(x, w0n_ref[...], preferred_element_type=jnp.float32)
    r0_ref[...] = (jnp.dot(x, w0r_ref[...], preferred_element_type=jnp.float32)
                   + jnp.dot(lip, w0l_ref[...], preferred_element_type=jnp.float32)
                   + b0_ref[...])
    l1_ref[...] = jnp.dot(lip, w1l_ref[...], preferred_element_type=jnp.float32)
    xsel = x[:, 0:1] * std_ref[0] + mu_ref[0]
    nz = (xsel != 0.0).astype(jnp.float32)
    col = lax.broadcasted_iota(jnp.int32, (BM, 16), 1)
    xs3 = jnp.where(col == 0, jnp.broadcast_to(xsel, (BM, 16)),
                    jnp.where(col == 1, jnp.broadcast_to(nz, (BM, 16)),
                              jnp.where(col == 2, 1.0, 0.0)))
    xs3_ref[...] = xs3


def _tc2_body(agga_ref, aggb_ref, r0_ref, w1n_ref, w1r_ref, l1_ref, b1_ref,
              y1_ref, r1_ref):
    agg = jnp.concatenate([agga_ref[...], aggb_ref[...]], axis=1)
    h0 = jnp.maximum(agg + r0_ref[...], 0.0)
    y1_ref[...] = jnp.dot(h0, w1n_ref[...], preferred_element_type=jnp.float32)
    r1_ref[...] = (jnp.dot(h0, w1r_ref[...], preferred_element_type=jnp.float32)
                   + l1_ref[...] + b1_ref[...])


def _tc3_body(agga_ref, aggb_ref, r1_ref, s0_ref, s1_ref, h_ref, sage_ref):
    agg = jnp.concatenate([agga_ref[...], aggb_ref[...]], axis=1)
    h_ref[...] = jnp.maximum(agg + r1_ref[...], 0.0)
    s = s0_ref[...] + s1_ref[...]
    d = jnp.maximum(s[:, 2:3], 1.0)
    sage_ref[...] = s / d


def _row_spec(width):
    return pl.BlockSpec((BM, width), lambda i: (i, 0))


def _full_spec(shape):
    nd = len(shape)
    return pl.BlockSpec(shape, lambda i: (0,) * nd)


_GRID = NP // BM

_tc1 = pl.pallas_call(
    _tc1_body,
    grid=(_GRID,),
    in_specs=[
        pl.BlockSpec(memory_space=pltpu.SMEM),
        pl.BlockSpec(memory_space=pltpu.SMEM),
        _row_spec(256), _row_spec(64),
        _full_spec((256, 256)), _full_spec((256, 256)), _full_spec((64, 256)),
        _full_spec((1, 256)), _full_spec((64, 256)),
    ],
    out_specs=[_row_spec(256), _row_spec(256), _row_spec(256), _row_spec(16)],
    out_shape=[
        jax.ShapeDtypeStruct((NP, 256), jnp.float32),
        jax.ShapeDtypeStruct((NP, 256), jnp.float32),
        jax.ShapeDtypeStruct((NP, 256), jnp.float32),
        jax.ShapeDtypeStruct((NP, 16), jnp.float32),
    ],
)

_tc2 = pl.pallas_call(
    _tc2_body,
    grid=(_GRID,),
    in_specs=[
        _row_spec(128), _row_spec(128), _row_spec(256),
        _full_spec((256, 256)), _full_spec((256, 256)),
        _row_spec(256), _full_spec((1, 256)),
    ],
    out_specs=[_row_spec(256), _row_spec(256)],
    out_shape=[
        jax.ShapeDtypeStruct((NP, 256), jnp.float32),
        jax.ShapeDtypeStruct((NP, 256), jnp.float32),
    ],
)

_tc3 = pl.pallas_call(
    _tc3_body,
    grid=(_GRID,),
    in_specs=[
        _row_spec(128), _row_spec(128), _row_spec(256),
        _row_spec(16), _row_spec(16),
    ],
    out_specs=[_row_spec(256), _row_spec(16)],
    out_shape=[
        jax.ShapeDtypeStruct((NP, 256), jnp.float32),
        jax.ShapeDtypeStruct((NP, 16), jnp.float32),
    ],
)


_EDGES_PER_TILE = EP // 16
_CHUNKS = _EDGES_PER_TILE // C
_SAGE_PER_TILE = EP // 32
_SAGE_CHUNKS = _SAGE_PER_TILE // C
_ZBL = 64
_ZBS = 128

_sc_mesh = plsc.VectorSubcoreMesh(core_axis_name="c", subcore_axis_name="s")


def _zero_vmem(buf, rows, width):
    z = jnp.zeros((16,), jnp.float32)
    per_row = width // 16

    def body(i, _):
        r = i // per_row
        j = i % per_row
        buf[r, pl.ds(j * 16, 16)] = z
        return 0

    lax.fori_loop(0, rows * per_row, body, 0)


def _edge_pass(src_h, dst_h, ws_h, y2_h, acc, src_v, dst_v, gidx2, dst2,
               w_v, rows_v, sem, c, s):
    def chunk(i, _):
        base = s * _EDGES_PER_TILE + i * C
        pltpu.sync_copy(src_h.at[pl.ds(base, C)], src_v)
        pltpu.sync_copy(dst_h.at[pl.ds(base, C)], dst_v)
        pltpu.sync_copy(ws_h.at[pl.ds(base, C)], w_v)
        for j in range(2):
            for k in range(4):
                sl = pl.ds(k * 16, 16)
                sv = pl.ds(j * 64 + k * 16, 16)
                gidx2[j, sl] = src_v[sv] * 2 + c
                dst2[j, sl] = dst_v[sv]
        for j in range(2):
            pltpu.async_copy(y2_h.at[gidx2.at[j]], rows_v, sem).wait()

            def scale(g, _):
                wv = w_v[pl.ds(j * 64 + g * 16, 16)]
                for i2 in range(16):
                    r = g * 16 + i2
                    wb = jnp.full((16,), wv[i2], jnp.float32)
                    for jj in range(8):
                        sl2 = pl.ds(jj * 16, 16)
                        rows_v[r, sl2] = rows_v[r, sl2] * wb
                return 0

            lax.fori_loop(0, 4, scale, 0)
            pltpu.sync_copy(rows_v, acc.at[dst2.at[j]], add=True)
        return 0

    lax.fori_loop(0, _CHUNKS, chunk, 0)


def _sage_pass(src_h, dst_h, t_h, t_acc, src_v, dst_v, gidx2, dst2,
               goff_s, goff_d, contrib, rows_v, sem, wid):
    zv = jnp.zeros((16,), jnp.float32)

    def chunk(i, _):
        base = wid * _SAGE_PER_TILE + i * C
        pltpu.sync_copy(src_h.at[pl.ds(base, C)], src_v)
        pltpu.sync_copy(dst_h.at[pl.ds(base, C)], dst_v)
        for j in range(2):
            for k in range(4):
                sl = pl.ds(k * 16, 16)
                sv = pl.ds(j * 64 + k * 16, 16)
                gidx2[j, sl] = jnp.right_shift(src_v[sv], 3)
                dst2[j, sl] = jnp.right_shift(dst_v[sv], 3)
                goff_s[sv] = jnp.bitwise_and(src_v[sv], 7) * 16
                goff_d[sv] = jnp.bitwise_and(dst_v[sv], 7) * 16
        for j in range(2):
            pltpu.async_copy(t_h.at[gidx2.at[j]], rows_v, sem).wait()

            def mv(g, _):
                off = pl.ds(j * 64 + g * 16, 16)
                vs = goff_s[off]
                vd = goff_d[off]
                for i2 in range(16):
                    r = g * 16 + i2
                    contrib[r, pl.ds(vd[i2], 16)] = rows_v[r, pl.ds(vs[i2], 16)]
                return 0

            lax.fori_loop(0, 4, mv, 0)
            pltpu.sync_copy(contrib, t_acc.at[dst2.at[j]], add=True)

            def rz(g, _):
                vd = goff_d[pl.ds(j * 64 + g * 16, 16)]
                for i2 in range(16):
                    contrib[g * 16 + i2, pl.ds(vd[i2], 16)] = zv
                return 0

            lax.fori_loop(0, 4, rz, 0)
        return 0

    lax.fori_loop(0, _SAGE_CHUNKS, chunk, 0)


_NSR = NP // 8


def _make_sc_kernel(do_sage):
    out_type = [jax.ShapeDtypeStruct((2 * NP, 128), jnp.float32)]
    scratch = [
        pltpu.VMEM_SHARED((NP, 128), jnp.float32),
        pltpu.VMEM((C,), jnp.int32),
        pltpu.VMEM((C,), jnp.int32),
        pltpu.VMEM((2, 64), jnp.int32),
        pltpu.VMEM((2, 64), jnp.int32),
        pltpu.VMEM((C,), jnp.float32),
        pltpu.VMEM((_ZBL, 128), jnp.float32),
        pltpu.SemaphoreType.DMA,
    ]
    if do_sage:
        out_type.append(jax.ShapeDtypeStruct((2 * _NSR, 128), jnp.float32))
        scratch += [
            pltpu.VMEM_SHARED((_NSR, 128), jnp.float32),
            pltpu.VMEM((C,), jnp.int32),
            pltpu.VMEM((C,), jnp.int32),
            pltpu.VMEM((_ZBL, 128), jnp.float32),
        ]

    def body(y2_h, src_h, dst_h, ws_h, *rest):
        if do_sage:
            (t_h, ssrc_h, sdst_h, aggL_out, sage_out, accL, src_v, dst_v,
             gidx2, dst2, w_v, rows_v, sem, t_acc, goff_s, goff_d,
             contrib) = rest
        else:
            (aggL_out, accL, src_v, dst_v, gidx2, dst2, w_v, rows_v,
             sem) = rest
        c = lax.axis_index("c")
        s = lax.axis_index("s")
        wid = c * 16 + s

        _zero_vmem(rows_v, _ZBL, 128)

        def zblkL(k, _):
            b = s + 16 * k
            pltpu.sync_copy(rows_v, accL.at[pl.ds(b * _ZBL, _ZBL)])
            return 0

        lax.fori_loop(0, NP // _ZBL // 16, zblkL, 0)
        if do_sage:
            _zero_vmem(contrib, _ZBL, 128)
            pltpu.sync_copy(rows_v, t_acc.at[pl.ds(s * 80, 64)])
            pltpu.sync_copy(rows_v.at[pl.ds(0, 16)],
                            t_acc.at[pl.ds(s * 80 + 64, 16)])
        plsc.subcore_barrier()

        _edge_pass(src_h, dst_h, ws_h, y2_h, accL, src_v, dst_v, gidx2,
                   dst2, w_v, rows_v, sem, c, s)
        if do_sage:
            _sage_pass(ssrc_h, sdst_h, t_h, t_acc, src_v, dst_v, gidx2,
                       dst2, goff_s, goff_d, contrib, rows_v, sem, wid)

        plsc.subcore_barrier()

        def oblkL(k, _):
            b = s + 16 * k
            pltpu.sync_copy(accL.at[pl.ds(b * _ZBL, _ZBL)],
                            aggL_out.at[pl.ds(c * NP + b * _ZBL, _ZBL)])
            return 0

        lax.fori_loop(0, NP // _ZBL // 16, oblkL, 0)
        if do_sage:
            pltpu.sync_copy(t_acc.at[pl.ds(s * 80, 80)],
                            sage_out.at[pl.ds(c * _NSR + s * 80, 80)])

    return pl.kernel(body, out_type=out_type, mesh=_sc_mesh,
                     scratch_types=scratch)


_sc1 = _make_sc_kernel(do_sage=True)
_sc2 = _make_sc_kernel(do_sage=False)


def _pad_edges(idx, eid, edge_weight):
    pad = EP - E
    src = jnp.concatenate([idx[0], jnp.zeros((pad,), jnp.int32)])
    dst = jnp.concatenate([idx[1], jnp.full((pad,), GARBAGE_ROW, jnp.int32)])
    ws = jnp.concatenate([jnp.take(edge_weight, eid),
                          jnp.zeros((pad,), jnp.float32)])
    return src, dst, ws


def kernel(xs, edge_index0, e_id0, edge_index1, e_id1, edge_weight,
           lipschitz, mu, std, W0n, W0r, W0l, b0, W1n, W1r, W1l, b1):
    x = jnp.pad(xs.reshape(N, 256), ((0, NP - N), (0, 0)))
    lip = jnp.pad(lipschitz, ((0, NP - N), (0, 0)))
    src0, dst0, ws0 = _pad_edges(edge_index0, e_id0, edge_weight)
    src1, dst1, ws1 = _pad_edges(edge_index1, e_id1, edge_weight)

    y0, r0, l1, xs3 = _tc1(mu.reshape(1), std.reshape(1), x, lip,
                           W0n, W0r, W0l, b0.reshape(1, 256), W1l)
    y0i = y0.reshape(2 * NP, 128)
    t_sage = xs3.reshape(_NSR, 128)

    agg0, sage_p = _sc1(y0i, src0, dst0, ws0, t_sage, src1, dst1)
    agg0 = agg0.reshape(2, NP, 128)
    sage_p = sage_p.reshape(2, _NSR, 128)
    s0 = sage_p[0].reshape(NP, 16)
    s1 = sage_p[1].reshape(NP, 16)

    y1, r1 = _tc2(agg0[0], agg0[1], r0, W1n, W1r, l1, b1.reshape(1, 256))
    y1i = y1.reshape(2 * NP, 128)

    agg1 = _sc2(y1i, src1, dst1, ws1)[0].reshape(2, NP, 128)

    h, sage = _tc3(agg1[0], agg1[1], r1, s0, s1)

    h_out = h[:N].reshape(1, 1, N, 256)
    x_sum = sage[:N, 0].reshape(1, 1, N, 1)
    count = sage[:N, 1].reshape(1, 1, N, 1)
    return (h_out, x_sum, count)


# speedup vs baseline: 3.2159x; 1.2587x over previous
import jax
import jax.numpy as jnp
from jax import lax
from jax.experimental import pallas as pl
from jax.experimental.pallas import tpu as pltpu
from jax.experimental.pallas import tpu_sc as plsc

N = 10000
E = 160000
NP = 10240
EP = 163840
GARBAGE_ROW = NP - 1
C = 128
BM = 256


def _tc1_body(mu_ref, std_ref, x_ref, lip_ref, w0n_ref, w0r_ref, w0l_ref,
              b0_ref, w1l_ref, y0_ref, r0_ref, l1_ref, xs3_ref):
    x = x_ref[...]
    lip = lip_ref[...]
    y0_ref[...] = jnp.dot(x, w0n_ref[...], preferred_element_type=jnp.float32)
    r0_ref[...] = (jnp.dot(x, w0r_ref[...], preferred_element_type=jnp.float32)
                   + jnp.dot(lip, w0l_ref[...], preferred_element_type=jnp.float32)
                   + b0_ref[...])
    l1_ref[...] = jnp.dot(lip, w1l_ref[...], preferred_element_type=jnp.float32)
    xsel = x[:, 0:1] * std_ref[0] + mu_ref[0]
    nz = (xsel != 0.0).astype(jnp.float32)
    col = lax.broadcasted_iota(jnp.int32, (BM, 16), 1)
    xs3 = jnp.where(col == 0, jnp.broadcast_to(xsel, (BM, 16)),
                    jnp.where(col == 1, jnp.broadcast_to(nz, (BM, 16)),
                              jnp.where(col == 2, 1.0, 0.0)))
    xs3_ref[...] = xs3


def _tc2_body(agga_ref, aggb_ref, r0_ref, w1n_ref, w1r_ref, l1_ref, b1_ref,
              y1_ref, r1_ref):
    agg = jnp.concatenate([agga_ref[...], aggb_ref[...]], axis=1)
    h0 = jnp.maximum(agg + r0_ref[...], 0.0)
    y1_ref[...] = jnp.dot(h0, w1n_ref[...], preferred_element_type=jnp.float32)
    r1_ref[...] = (jnp.dot(h0, w1r_ref[...], preferred_element_type=jnp.float32)
                   + l1_ref[...] + b1_ref[...])


def _tc3_body(agga_ref, aggb_ref, r1_ref, s0_ref, s1_ref, h_ref, sage_ref):
    agg = jnp.concatenate([agga_ref[...], aggb_ref[...]], axis=1)
    h_ref[...] = jnp.maximum(agg + r1_ref[...], 0.0)
    s = s0_ref[...] + s1_ref[...]
    d = jnp.maximum(s[:, 2:3], 1.0)
    sage_ref[...] = s / d


def _row_spec(width):
    return pl.BlockSpec((BM, width), lambda i: (i, 0))


def _full_spec(shape):
    nd = len(shape)
    return pl.BlockSpec(shape, lambda i: (0,) * nd)


_GRID = NP // BM

_tc1 = pl.pallas_call(
    _tc1_body,
    grid=(_GRID,),
    in_specs=[
        pl.BlockSpec(memory_space=pltpu.SMEM),
        pl.BlockSpec(memory_space=pltpu.SMEM),
        _row_spec(256), _row_spec(64),
        _full_spec((256, 256)), _full_spec((256, 256)), _full_spec((64, 256)),
        _full_spec((1, 256)), _full_spec((64, 256)),
    ],
    out_specs=[_row_spec(256), _row_spec(256), _row_spec(256), _row_spec(16)],
    out_shape=[
        jax.ShapeDtypeStruct((NP, 256), jnp.float32),
        jax.ShapeDtypeStruct((NP, 256), jnp.float32),
        jax.ShapeDtypeStruct((NP, 256), jnp.float32),
        jax.ShapeDtypeStruct((NP, 16), jnp.float32),
    ],
)

_tc2 = pl.pallas_call(
    _tc2_body,
    grid=(_GRID,),
    in_specs=[
        _row_spec(128), _row_spec(128), _row_spec(256),
        _full_spec((256, 256)), _full_spec((256, 256)),
        _row_spec(256), _full_spec((1, 256)),
    ],
    out_specs=[_row_spec(256), _row_spec(256)],
    out_shape=[
        jax.ShapeDtypeStruct((NP, 256), jnp.float32),
        jax.ShapeDtypeStruct((NP, 256), jnp.float32),
    ],
)

_tc3 = pl.pallas_call(
    _tc3_body,
    grid=(_GRID,),
    in_specs=[
        _row_spec(128), _row_spec(128), _row_spec(256),
        _row_spec(16), _row_spec(16),
    ],
    out_specs=[_row_spec(256), _row_spec(16)],
    out_shape=[
        jax.ShapeDtypeStruct((NP, 256), jnp.float32),
        jax.ShapeDtypeStruct((NP, 16), jnp.float32),
    ],
)


_EDGES_PER_TILE = EP // 16
_CHUNKS = _EDGES_PER_TILE // C
_SAGE_PER_TILE = EP // 32
_SAGE_CHUNKS = _SAGE_PER_TILE // C
_ZBL = 64
_ZBS = 128

_sc_mesh = plsc.VectorSubcoreMesh(core_axis_name="c", subcore_axis_name="s")


def _zero_vmem(buf, rows, width):
    z = jnp.zeros((16,), jnp.float32)
    per_row = width // 16

    def body(i, _):
        r = i // per_row
        j = i % per_row
        buf[r, pl.ds(j * 16, 16)] = z
        return 0

    lax.fori_loop(0, rows * per_row, body, 0)


def _edge_pass(src_h, dst_h, ws_h, y2_h, acc, srcb, dstb, wb, gidx2,
               dst2, rows, semi, semg, sems, c, s):
    def issue_idx(i_chunk, pb):
        base = s * _EDGES_PER_TILE + i_chunk * C
        pltpu.async_copy(src_h.at[pl.ds(base, C)], srcb[pb], semi[pb])
        pltpu.async_copy(dst_h.at[pl.ds(base, C)], dstb[pb], semi[pb])
        pltpu.async_copy(ws_h.at[pl.ds(base, C)], wb[pb], semi[pb])

    def wait_idx(pb):
        pltpu.make_async_copy(src_h.at[pl.ds(0, C)], srcb[pb], semi[pb]).wait()
        pltpu.make_async_copy(dst_h.at[pl.ds(0, C)], dstb[pb], semi[pb]).wait()
        pltpu.make_async_copy(ws_h.at[pl.ds(0, C)], wb[pb], semi[pb]).wait()

    def wait_scatter(q):
        pltpu.make_async_copy(rows[q], acc.at[pl.ds(0, 64)], sems[q]).wait()

    def wait_gather(q):
        pltpu.make_async_copy(y2_h.at[pl.ds(0, 64)], rows[q], semg[q]).wait()

    issue_idx(0, 0)
    nsuper = _CHUNKS // 2

    def superchunk(g, _):
        for pb in range(2):
            i = 2 * g + pb
            wait_idx(pb)
            for j in range(2):
                for k in range(4):
                    sl = pl.ds(k * 16, 16)
                    sv = pl.ds(j * 64 + k * 16, 16)
                    gidx2[pb][j, sl] = srcb[pb][sv] * 2 + c
                    dst2[pb][j, sl] = dstb[pb][sv]
            if pb == 0:
                @pl.when(g > 0)
                def _w0():
                    wait_scatter(0)
            else:
                wait_scatter(0)
            pltpu.async_copy(y2_h.at[gidx2[pb].at[0]], rows[0], semg[0])
            if pb == 0:
                issue_idx(i + 1, 1)
            else:
                @pl.when(g < nsuper - 1)
                def _wi():
                    issue_idx(i + 1, 0)
            for q in range(2):
                wait_gather(q)
                if q == 0:
                    if pb == 0:
                        @pl.when(g > 0)
                        def _w1():
                            wait_scatter(1)
                    else:
                        wait_scatter(1)
                    pltpu.async_copy(y2_h.at[gidx2[pb].at[1]], rows[1], semg[1])

                def scale(gg, _):
                    wv = wb[pb][pl.ds(q * 64 + gg * 16, 16)]
                    for i2 in range(16):
                        r = gg * 16 + i2
                        wbv = jnp.full((16,), wv[i2], jnp.float32)
                        for jj in range(8):
                            sl2 = pl.ds(jj * 16, 16)
                            rows[q][r, sl2] = rows[q][r, sl2] * wbv
                    return 0

                lax.fori_loop(0, 4, scale, 0)
                pltpu.async_copy(rows[q], acc.at[dst2[pb].at[q]], sems[q],
                                 add=True)
        return 0

    lax.fori_loop(0, nsuper, superchunk, 0)
    wait_scatter(0)
    wait_scatter(1)


def _sage_pass(src_h, dst_h, t_h, t_acc, src_v, dst_v, gidx2, dst2,
               goff_s, goff_d, contrib, rows_v, sem, wid):
    zv = jnp.zeros((16,), jnp.float32)

    def chunk(i, _):
        base = wid * _SAGE_PER_TILE + i * C
        pltpu.sync_copy(src_h.at[pl.ds(base, C)], src_v)
        pltpu.sync_copy(dst_h.at[pl.ds(base, C)], dst_v)
        for j in range(2):
            for k in range(4):
                sl = pl.ds(k * 16, 16)
                sv = pl.ds(j * 64 + k * 16, 16)
                gidx2[j, sl] = jnp.right_shift(src_v[sv], 3)
                dst2[j, sl] = jnp.right_shift(dst_v[sv], 3)
                goff_s[sv] = jnp.bitwise_and(src_v[sv], 7) * 16
                goff_d[sv] = jnp.bitwise_and(dst_v[sv], 7) * 16
        for j in range(2):
            pltpu.async_copy(t_h.at[gidx2.at[j]], rows_v, sem).wait()

            def mv(g, _):
                off = pl.ds(j * 64 + g * 16, 16)
                vs = goff_s[off]
                vd = goff_d[off]
                for i2 in range(16):
                    r = g * 16 + i2
                    contrib[r, pl.ds(vd[i2], 16)] = rows_v[r, pl.ds(vs[i2], 16)]
                return 0

            lax.fori_loop(0, 4, mv, 0)
            pltpu.sync_copy(contrib, t_acc.at[dst2.at[j]], add=True)

            def rz(g, _):
                vd = goff_d[pl.ds(j * 64 + g * 16, 16)]
                for i2 in range(16):
                    contrib[g * 16 + i2, pl.ds(vd[i2], 16)] = zv
                return 0

            lax.fori_loop(0, 4, rz, 0)
        return 0

    lax.fori_loop(0, _SAGE_CHUNKS, chunk, 0)


_NSR = NP // 8


def _make_sc_kernel(do_sage):
    out_type = [jax.ShapeDtypeStruct((2 * NP, 128), jnp.float32)]
    scratch = [
        pltpu.VMEM_SHARED((NP, 128), jnp.float32),
        pltpu.VMEM((C,), jnp.int32),
        pltpu.VMEM((C,), jnp.int32),
        pltpu.VMEM((C,), jnp.int32),
        pltpu.VMEM((C,), jnp.int32),
        pltpu.VMEM((C,), jnp.float32),
        pltpu.VMEM((C,), jnp.float32),
        pltpu.VMEM((2, 64), jnp.int32),
        pltpu.VMEM((2, 64), jnp.int32),
        pltpu.VMEM((2, 64), jnp.int32),
        pltpu.VMEM((2, 64), jnp.int32),
        pltpu.VMEM((_ZBL, 128), jnp.float32),
        pltpu.VMEM((_ZBL, 128), jnp.float32),
        pltpu.SemaphoreType.DMA,
        pltpu.SemaphoreType.DMA,
        pltpu.SemaphoreType.DMA,
        pltpu.SemaphoreType.DMA,
        pltpu.SemaphoreType.DMA,
        pltpu.SemaphoreType.DMA,
        pltpu.SemaphoreType.DMA,
    ]
    if do_sage:
        out_type.append(jax.ShapeDtypeStruct((2 * _NSR, 128), jnp.float32))
        scratch += [
            pltpu.VMEM_SHARED((_NSR, 128), jnp.float32),
            pltpu.VMEM((C,), jnp.int32),
            pltpu.VMEM((C,), jnp.int32),
        ]

    def body(y2_h, src_h, dst_h, ws_h, *rest):
        if do_sage:
            (t_h, ssrc_h, sdst_h, aggL_out, sage_out, accL, src_a, src_b,
             dst_a, dst_b, w_a, w_b, gidx2a, gidx2b, dst2a, dst2b, rows0,
             rows1, semi0, semi1, semg0, semg1, sems0, sems1, semx, t_acc,
             goff_s, goff_d) = rest
        else:
            (aggL_out, accL, src_a, src_b, dst_a, dst_b, w_a, w_b, gidx2a,
             gidx2b, dst2a, dst2b, rows0, rows1, semi0, semi1, semg0,
             semg1, sems0, sems1, semx) = rest
        c = lax.axis_index("c")
        s = lax.axis_index("s")
        wid = c * 16 + s

        _zero_vmem(rows0, _ZBL, 128)

        def zblkL(k, _):
            b = s + 16 * k
            pltpu.sync_copy(rows0, accL.at[pl.ds(b * _ZBL, _ZBL)])
            return 0

        lax.fori_loop(0, NP // _ZBL // 16, zblkL, 0)
        if do_sage:
            pltpu.sync_copy(rows0, t_acc.at[pl.ds(s * 80, 64)])
            pltpu.sync_copy(rows0.at[pl.ds(0, 16)],
                            t_acc.at[pl.ds(s * 80 + 64, 16)])
        plsc.subcore_barrier()

        _edge_pass(src_h, dst_h, ws_h, y2_h, accL, [src_a, src_b],
                   [dst_a, dst_b], [w_a, w_b], [gidx2a, gidx2b],
                   [dst2a, dst2b], [rows0, rows1],
                   [semi0, semi1], [semg0, semg1], [sems0, sems1], c, s)
        if do_sage:
            _zero_vmem(rows1, _ZBL, 128)
            _sage_pass(ssrc_h, sdst_h, t_h, t_acc, src_a, dst_a, gidx2a,
                       dst2a, goff_s, goff_d, rows1, rows0, semx, wid)

        plsc.subcore_barrier()

        def oblkL(k, _):
            b = s + 16 * k
            pltpu.sync_copy(accL.at[pl.ds(b * _ZBL, _ZBL)],
                            aggL_out.at[pl.ds(c * NP + b * _ZBL, _ZBL)])
            return 0

        lax.fori_loop(0, NP // _ZBL // 16, oblkL, 0)
        if do_sage:
            pltpu.sync_copy(t_acc.at[pl.ds(s * 80, 80)],
                            sage_out.at[pl.ds(c * _NSR + s * 80, 80)])

    return pl.kernel(body, out_type=out_type, mesh=_sc_mesh,
                     scratch_types=scratch)


_sc1 = _make_sc_kernel(do_sage=True)
_sc2 = _make_sc_kernel(do_sage=False)


def _pad_edges(idx, eid, edge_weight):
    pad = EP - E
    src = jnp.concatenate([idx[0], jnp.zeros((pad,), jnp.int32)])
    dst = jnp.concatenate([idx[1], jnp.full((pad,), GARBAGE_ROW, jnp.int32)])
    ws = jnp.concatenate([jnp.take(edge_weight, eid),
                          jnp.zeros((pad,), jnp.float32)])
    return src, dst, ws


def kernel(xs, edge_index0, e_id0, edge_index1, e_id1, edge_weight,
           lipschitz, mu, std, W0n, W0r, W0l, b0, W1n, W1r, W1l, b1):
    x = jnp.pad(xs.reshape(N, 256), ((0, NP - N), (0, 0)))
    lip = jnp.pad(lipschitz, ((0, NP - N), (0, 0)))
    src0, dst0, ws0 = _pad_edges(edge_index0, e_id0, edge_weight)
    src1, dst1, ws1 = _pad_edges(edge_index1, e_id1, edge_weight)

    y0, r0, l1, xs3 = _tc1(mu.reshape(1), std.reshape(1), x, lip,
                           W0n, W0r, W0l, b0.reshape(1, 256), W1l)
    y0i = y0.reshape(2 * NP, 128)
    t_sage = xs3.reshape(_NSR, 128)

    agg0, sage_p = _sc1(y0i, src0, dst0, ws0, t_sage, src1, dst1)
    agg0 = agg0.reshape(2, NP, 128)
    sage_p = sage_p.reshape(2, _NSR, 128)
    s0 = sage_p[0].reshape(NP, 16)
    s1 = sage_p[1].reshape(NP, 16)

    y1, r1 = _tc2(agg0[0], agg0[1], r0, W1n, W1r, l1, b1.reshape(1, 256))
    y1i = y1.reshape(2 * NP, 128)

    agg1 = _sc2(y1i, src1, dst1, ws1)[0].reshape(2, NP, 128)

    h, sage = _tc3(agg1[0], agg1[1], r1, s0, s1)

    h_out = h[:N].reshape(1, 1, N, 256)
    x_sum = sage[:N, 0].reshape(1, 1, N, 1)
    count = sage[:N, 1].reshape(1, 1, N, 1)
    return (h_out, x_sum, count)


# speedup vs baseline: 3.3207x; 1.0326x over previous
import jax
import jax.numpy as jnp
from jax import lax
from jax.experimental import pallas as pl
from jax.experimental.pallas import tpu as pltpu
from jax.experimental.pallas import tpu_sc as plsc

N = 10000
E = 160000
NP = 10240
EP = 163840
GARBAGE_ROW = NP - 1
C = 128
BM = 256


def _tc1_body(mu_ref, std_ref, x_ref, lip_ref, w0n_ref, w0r_ref, w0l_ref,
              b0_ref, w1l_ref, y0_ref, r0_ref, l1_ref, xs3_ref):
    x = x_ref[...]
    lip = lip_ref[...]
    y0_ref[...] = jnp.dot(x, w0n_ref[...], preferred_element_type=jnp.float32)
    r0_ref[...] = (jnp.dot(x, w0r_ref[...], preferred_element_type=jnp.float32)
                   + jnp.dot(lip, w0l_ref[...], preferred_element_type=jnp.float32)
                   + b0_ref[...])
    l1_ref[...] = jnp.dot(lip, w1l_ref[...], preferred_element_type=jnp.float32)
    xsel = x[:, 0:1] * std_ref[0] + mu_ref[0]
    nz = (xsel != 0.0).astype(jnp.float32)
    col = lax.broadcasted_iota(jnp.int32, (BM, 16), 1)
    xs3 = jnp.where(col == 0, jnp.broadcast_to(xsel, (BM, 16)),
                    jnp.where(col == 1, jnp.broadcast_to(nz, (BM, 16)),
                              jnp.where(col == 2, 1.0, 0.0)))
    xs3_ref[...] = xs3


def _tc2_body(agga_ref, aggb_ref, r0_ref, w1n_ref, w1r_ref, l1_ref, b1_ref,
              y1_ref, r1_ref):
    agg = jnp.concatenate([agga_ref[...], aggb_ref[...]], axis=1)
    h0 = jnp.maximum(agg + r0_ref[...], 0.0)
    y1_ref[...] = jnp.dot(h0, w1n_ref[...], preferred_element_type=jnp.float32)
    r1_ref[...] = (jnp.dot(h0, w1r_ref[...], preferred_element_type=jnp.float32)
                   + l1_ref[...] + b1_ref[...])


def _tc3_body(agga_ref, aggb_ref, r1_ref, s0_ref, s1_ref, h_ref, sage_ref):
    agg = jnp.concatenate([agga_ref[...], aggb_ref[...]], axis=1)
    h_ref[...] = jnp.maximum(agg + r1_ref[...], 0.0)
    s = s0_ref[...] + s1_ref[...]
    d = jnp.maximum(s[:, 2:3], 1.0)
    sage_ref[...] = s / d


def _row_spec(width):
    return pl.BlockSpec((BM, width), lambda i: (i, 0))


def _full_spec(shape):
    nd = len(shape)
    return pl.BlockSpec(shape, lambda i: (0,) * nd)


_GRID = NP // BM

_tc1 = pl.pallas_call(
    _tc1_body,
    grid=(_GRID,),
    in_specs=[
        pl.BlockSpec(memory_space=pltpu.SMEM),
        pl.BlockSpec(memory_space=pltpu.SMEM),
        _row_spec(256), _row_spec(64),
        _full_spec((256, 256)), _full_spec((256, 256)), _full_spec((64, 256)),
        _full_spec((1, 256)), _full_spec((64, 256)),
    ],
    out_specs=[_row_spec(256), _row_spec(256), _row_spec(256), _row_spec(16)],
    out_shape=[
        jax.ShapeDtypeStruct((NP, 256), jnp.float32),
        jax.ShapeDtypeStruct((NP, 256), jnp.float32),
        jax.ShapeDtypeStruct((NP, 256), jnp.float32),
        jax.ShapeDtypeStruct((NP, 16), jnp.float32),
    ],
)

_tc2 = pl.pallas_call(
    _tc2_body,
    grid=(_GRID,),
    in_specs=[
        _row_spec(128), _row_spec(128), _row_spec(256),
        _full_spec((256, 256)), _full_spec((256, 256)),
        _row_spec(256), _full_spec((1, 256)),
    ],
    out_specs=[_row_spec(256), _row_spec(256)],
    out_shape=[
        jax.ShapeDtypeStruct((NP, 256), jnp.float32),
        jax.ShapeDtypeStruct((NP, 256), jnp.float32),
    ],
)

_tc3 = pl.pallas_call(
    _tc3_body,
    grid=(_GRID,),
    in_specs=[
        _row_spec(128), _row_spec(128), _row_spec(256),
        _row_spec(16), _row_spec(16),
    ],
    out_specs=[_row_spec(256), _row_spec(16)],
    out_shape=[
        jax.ShapeDtypeStruct((NP, 256), jnp.float32),
        jax.ShapeDtypeStruct((NP, 16), jnp.float32),
    ],
)


_EDGES_PER_TILE = EP // 16
_CHUNKS = _EDGES_PER_TILE // C
_SAGE_PER_TILE = EP // 32
_SAGE_CHUNKS = _SAGE_PER_TILE // C
_ZBL = 64
_ZBS = 128

_sc_mesh = plsc.VectorSubcoreMesh(core_axis_name="c", subcore_axis_name="s")


def _zero_vmem(buf, rows, width):
    z = jnp.zeros((16,), jnp.float32)
    per_row = width // 16

    @plsc.parallel_loop(0, rows * per_row, unroll=8)
    def body(i):
        r = i // per_row
        j = i % per_row
        buf[r, pl.ds(j * 16, 16)] = z


def _edge_pass(src_h, dst_h, ws_h, y2_h, acc, srcb, dstb, wb, gidx2,
               dst2, rows, semi, semg, sems, c, s):
    def issue_idx(i_chunk, pb):
        base = s * _EDGES_PER_TILE + i_chunk * C
        pltpu.async_copy(src_h.at[pl.ds(base, C)], srcb[pb], semi[pb])
        pltpu.async_copy(dst_h.at[pl.ds(base, C)], dstb[pb], semi[pb])
        pltpu.async_copy(ws_h.at[pl.ds(base, C)], wb[pb], semi[pb])

    def wait_idx(pb):
        pltpu.make_async_copy(src_h.at[pl.ds(0, C)], srcb[pb], semi[pb]).wait()
        pltpu.make_async_copy(dst_h.at[pl.ds(0, C)], dstb[pb], semi[pb]).wait()
        pltpu.make_async_copy(ws_h.at[pl.ds(0, C)], wb[pb], semi[pb]).wait()

    def wait_scatter(q):
        pltpu.make_async_copy(rows[q], acc.at[pl.ds(0, 64)], sems[q]).wait()

    def wait_gather(q):
        pltpu.make_async_copy(y2_h.at[pl.ds(0, 64)], rows[q], semg[q]).wait()

    issue_idx(0, 0)
    nsuper = _CHUNKS // 2

    def superchunk(g, _):
        for pb in range(2):
            i = 2 * g + pb
            wait_idx(pb)
            for j in range(2):
                for k in range(4):
                    sl = pl.ds(k * 16, 16)
                    sv = pl.ds(j * 64 + k * 16, 16)
                    gidx2[pb][j, sl] = srcb[pb][sv] * 2 + c
                    dst2[pb][j, sl] = dstb[pb][sv]
            if pb == 0:
                @pl.when(g > 0)
                def _w0():
                    wait_scatter(0)
            else:
                wait_scatter(0)
            pltpu.async_copy(y2_h.at[gidx2[pb].at[0]], rows[0], semg[0])
            if pb == 0:
                issue_idx(i + 1, 1)
            else:
                @pl.when(g < nsuper - 1)
                def _wi():
                    issue_idx(i + 1, 0)
            for q in range(2):
                wait_gather(q)
                if q == 0:
                    if pb == 0:
                        @pl.when(g > 0)
                        def _w1():
                            wait_scatter(1)
                    else:
                        wait_scatter(1)
                    pltpu.async_copy(y2_h.at[gidx2[pb].at[1]], rows[1], semg[1])

                @plsc.parallel_loop(0, 4, unroll=2)
                def scale(gg):
                    wv = wb[pb][pl.ds(q * 64 + gg * 16, 16)]
                    for i2 in range(16):
                        r = gg * 16 + i2
                        wbv = jnp.full((16,), wv[i2], jnp.float32)
                        for jj in range(8):
                            sl2 = pl.ds(jj * 16, 16)
                            rows[q][r, sl2] = rows[q][r, sl2] * wbv
                pltpu.async_copy(rows[q], acc.at[dst2[pb].at[q]], sems[q],
                                 add=True)
        return 0

    lax.fori_loop(0, nsuper, superchunk, 0)
    wait_scatter(0)
    wait_scatter(1)


def _sage_pass(src_h, dst_h, t_h, t_acc, src_v, dst_v, gidx4, dst4,
               goff_s, goff_d, contrib, rows_v, semg, wid):
    zv = jnp.zeros((16,), jnp.float32)
    halves = [rows_v.at[pl.ds(0, 32)], rows_v.at[pl.ds(32, 32)]]
    chalves = [contrib.at[pl.ds(0, 32)], contrib.at[pl.ds(32, 32)]]

    def chunk(i, _):
        base = wid * _SAGE_PER_TILE + i * C
        pltpu.sync_copy(src_h.at[pl.ds(base, C)], src_v)
        pltpu.sync_copy(dst_h.at[pl.ds(base, C)], dst_v)
        for q in range(4):
            for k in range(2):
                sl = pl.ds(k * 16, 16)
                sv = pl.ds(q * 32 + k * 16, 16)
                gidx4[q, sl] = jnp.right_shift(src_v[sv], 3)
                dst4[q, sl] = jnp.right_shift(dst_v[sv], 3)
                goff_s[sv] = jnp.bitwise_and(src_v[sv], 7) * 16
                goff_d[sv] = jnp.bitwise_and(dst_v[sv], 7) * 16
        pltpu.async_copy(t_h.at[gidx4.at[0]], halves[0], semg[0])
        pltpu.async_copy(t_h.at[gidx4.at[1]], halves[1], semg[1])
        for q in range(4):
            hq = q % 2
            pltpu.make_async_copy(t_h.at[pl.ds(0, 32)], halves[hq],
                                  semg[hq]).wait()

            @plsc.parallel_loop(0, 2)
            def mv(g):
                off = pl.ds(q * 32 + g * 16, 16)
                vs = goff_s[off]
                vd = goff_d[off]
                for i2 in range(16):
                    r = hq * 32 + g * 16 + i2
                    contrib[r, pl.ds(vd[i2], 16)] = \
                        rows_v[r, pl.ds(vs[i2], 16)]

            if q < 2:
                pltpu.async_copy(t_h.at[gidx4.at[q + 2]], halves[hq],
                                 semg[hq])
            pltpu.sync_copy(chalves[hq], t_acc.at[dst4.at[q]], add=True)

            @plsc.parallel_loop(0, 2)
            def rz(g):
                vd = goff_d[pl.ds(q * 32 + g * 16, 16)]
                for i2 in range(16):
                    contrib[hq * 32 + g * 16 + i2, pl.ds(vd[i2], 16)] = zv

        return 0

    lax.fori_loop(0, _SAGE_CHUNKS, chunk, 0)


_NSR = NP // 8


def _make_sc_kernel(do_sage):
    out_type = [jax.ShapeDtypeStruct((2 * NP, 128), jnp.float32)]
    scratch = [
        pltpu.VMEM_SHARED((NP, 128), jnp.float32),
        pltpu.VMEM((C,), jnp.int32),
        pltpu.VMEM((C,), jnp.int32),
        pltpu.VMEM((C,), jnp.int32),
        pltpu.VMEM((C,), jnp.int32),
        pltpu.VMEM((C,), jnp.float32),
        pltpu.VMEM((C,), jnp.float32),
        pltpu.VMEM((2, 64), jnp.int32),
        pltpu.VMEM((2, 64), jnp.int32),
        pltpu.VMEM((2, 64), jnp.int32),
        pltpu.VMEM((2, 64), jnp.int32),
        pltpu.VMEM((_ZBL, 128), jnp.float32),
        pltpu.VMEM((_ZBL, 128), jnp.float32),
        pltpu.SemaphoreType.DMA,
        pltpu.SemaphoreType.DMA,
        pltpu.SemaphoreType.DMA,
        pltpu.SemaphoreType.DMA,
        pltpu.SemaphoreType.DMA,
        pltpu.SemaphoreType.DMA,
        pltpu.SemaphoreType.DMA,
    ]
    if do_sage:
        out_type.append(jax.ShapeDtypeStruct((2 * _NSR, 128), jnp.float32))
        scratch += [
            pltpu.VMEM_SHARED((_NSR, 128), jnp.float32),
            pltpu.VMEM((C,), jnp.int32),
            pltpu.VMEM((C,), jnp.int32),
            pltpu.VMEM((4, 32), jnp.int32),
            pltpu.VMEM((4, 32), jnp.int32),
        ]

    def body(y2_h, src_h, dst_h, ws_h, *rest):
        if do_sage:
            (t_h, ssrc_h, sdst_h, aggL_out, sage_out, accL, src_a, src_b,
             dst_a, dst_b, w_a, w_b, gidx2a, gidx2b, dst2a, dst2b, rows0,
             rows1, semi0, semi1, semg0, semg1, sems0, sems1, semx, t_acc,
             goff_s, goff_d, gidx4, dst4) = rest
        else:
            (aggL_out, accL, src_a, src_b, dst_a, dst_b, w_a, w_b, gidx2a,
             gidx2b, dst2a, dst2b, rows0, rows1, semi0, semi1, semg0,
             semg1, sems0, sems1, semx) = rest
        c = lax.axis_index("c")
        s = lax.axis_index("s")
        wid = c * 16 + s

        _zero_vmem(rows0, _ZBL, 128)

        def zblkL(k, _):
            b = s + 16 * k
            pltpu.sync_copy(rows0, accL.at[pl.ds(b * _ZBL, _ZBL)])
            return 0

        lax.fori_loop(0, NP // _ZBL // 16, zblkL, 0)
        if do_sage:
            pltpu.sync_copy(rows0, t_acc.at[pl.ds(s * 80, 64)])
            pltpu.sync_copy(rows0.at[pl.ds(0, 16)],
                            t_acc.at[pl.ds(s * 80 + 64, 16)])
        plsc.subcore_barrier()

        _edge_pass(src_h, dst_h, ws_h, y2_h, accL, [src_a, src_b],
                   [dst_a, dst_b], [w_a, w_b], [gidx2a, gidx2b],
                   [dst2a, dst2b], [rows0, rows1],
                   [semi0, semi1], [semg0, semg1], [sems0, sems1], c, s)
        if do_sage:
            _zero_vmem(rows1, _ZBL, 128)
            _sage_pass(ssrc_h, sdst_h, t_h, t_acc, src_a, dst_a, gidx4,
                       dst4, goff_s, goff_d, rows1, rows0,
                       [semg0, semg1], wid)

        plsc.subcore_barrier()

        def oblkL(k, _):
            b = s + 16 * k
            pltpu.sync_copy(accL.at[pl.ds(b * _ZBL, _ZBL)],
                            aggL_out.at[pl.ds(c * NP + b * _ZBL, _ZBL)])
            return 0

        lax.fori_loop(0, NP // _ZBL // 16, oblkL, 0)
        if do_sage:
            pltpu.sync_copy(t_acc.at[pl.ds(s * 80, 80)],
                            sage_out.at[pl.ds(c * _NSR + s * 80, 80)])

    return pl.kernel(body, out_type=out_type, mesh=_sc_mesh,
                     scratch_types=scratch)


_sc1 = _make_sc_kernel(do_sage=True)
_sc2 = _make_sc_kernel(do_sage=False)


def _pad_edges(idx, eid, edge_weight):
    pad = EP - E
    src = jnp.concatenate([idx[0], jnp.zeros((pad,), jnp.int32)])
    dst = jnp.concatenate([idx[1], jnp.full((pad,), GARBAGE_ROW, jnp.int32)])
    ws = jnp.concatenate([jnp.take(edge_weight, eid),
                          jnp.zeros((pad,), jnp.float32)])
    return src, dst, ws


def kernel(xs, edge_index0, e_id0, edge_index1, e_id1, edge_weight,
           lipschitz, mu, std, W0n, W0r, W0l, b0, W1n, W1r, W1l, b1):
    x = jnp.pad(xs.reshape(N, 256), ((0, NP - N), (0, 0)))
    lip = jnp.pad(lipschitz, ((0, NP - N), (0, 0)))
    src0, dst0, ws0 = _pad_edges(edge_index0, e_id0, edge_weight)
    src1, dst1, ws1 = _pad_edges(edge_index1, e_id1, edge_weight)

    y0, r0, l1, xs3 = _tc1(mu.reshape(1), std.reshape(1), x, lip,
                           W0n, W0r, W0l, b0.reshape(1, 256), W1l)
    y0i = y0.reshape(2 * NP, 128)
    t_sage = xs3.reshape(_NSR, 128)

    agg0, sage_p = _sc1(y0i, src0, dst0, ws0, t_sage, src1, dst1)
    agg0 = agg0.reshape(2, NP, 128)
    sage_p = sage_p.reshape(2, _NSR, 128)
    s0 = sage_p[0].reshape(NP, 16)
    s1 = sage_p[1].reshape(NP, 16)

    y1, r1 = _tc2(agg0[0], agg0[1], r0, W1n, W1r, l1, b1.reshape(1, 256))
    y1i = y1.reshape(2 * NP, 128)

    agg1 = _sc2(y1i, src1, dst1, ws1)[0].reshape(2, NP, 128)

    h, sage = _tc3(agg1[0], agg1[1], r1, s0, s1)

    h_out = h[:N].reshape(1, 1, N, 256)
    x_sum = sage[:N, 0].reshape(1, 1, N, 1)
    count = sage[:N, 1].reshape(1, 1, N, 1)
    return (h_out, x_sum, count)
